# initial kernel scaffold (unmeasured)
import jax
import jax.numpy as jnp
from jax import lax
from jax.experimental import pallas as pl
from jax.experimental.pallas import tpu as pltpu

N_DEV = 4
M_GLOBAL = 8192
K = 8192
N_SHARD = 1024
M_SHARD = 2048
HALF = 1024
BLK = 512

CW, CCW = 0, 1


def kernel(x, w_mat):
    assert x.shape == (M_SHARD, K), x.shape
    assert w_mat.shape == (K, N_SHARD), w_mat.shape

    def body(x_ref, w_ref, out_ref, xg_ref, a_stage, o_stage,
             send_sems, recv_sems, cp_in_sem, cp_out_sem):
        p = lax.axis_index("i")
        left = lax.rem(p + N_DEV - 1, N_DEV)
        right = lax.rem(p + 1, N_DEV)

        barrier = pltpu.get_barrier_semaphore()
        for nbr in (left, right):
            pl.semaphore_signal(
                barrier, inc=1,
                device_id=(nbr,), device_id_type=pl.DeviceIdType.MESH,
            )
        pl.semaphore_wait(barrier, 2)

        def chunk_gemm(src_ref, src_base, out_base, nblk):
            def blk(b, carry):
                sr = src_base + b * BLK
                orow = out_base + b * BLK
                cp = pltpu.make_async_copy(
                    src_ref.at[pl.ds(sr, BLK), :], a_stage, cp_in_sem)
                cp.start()
                cp.wait()
                o_stage[...] = jnp.maximum(
                    jnp.dot(a_stage[...], w_ref[...],
                            preferred_element_type=jnp.float32),
                    0.0)
                cp2 = pltpu.make_async_copy(
                    o_stage, out_ref.at[pl.ds(orow, BLK), :], cp_out_sem)
                cp2.start()
                cp2.wait()
                return carry
            lax.fori_loop(0, nblk, blk, 0)

        def remote_copy(src_ref, rows, d, h, target):
            return pltpu.make_async_remote_copy(
                src_ref=src_ref,
                dst_ref=xg_ref.at[pl.ds(rows, HALF), :],
                send_sem=send_sems.at[d, h],
                recv_sem=recv_sems.at[d, h],
                device_id=(target,),
                device_id_type=pl.DeviceIdType.MESH,
            )

        sends = []

        cw0 = remote_copy(x_ref.at[pl.ds(0, HALF), :],
                          p * M_SHARD, CW, 0, right)
        ccw0 = remote_copy(x_ref.at[pl.ds(HALF, HALF), :],
                           p * M_SHARD + HALF, CCW, 0, left)
        cw0.start()
        ccw0.start()
        sends += [cw0, ccw0]

        chunk_gemm(x_ref, 0, p * M_SHARD, M_SHARD // BLK)

        for h in range(N_DEV - 1):
            q_cw = lax.rem(p + N_DEV - 1 - h, N_DEV)
            q_ccw = lax.rem(p + 1 + h, N_DEV)
            rows_cw = q_cw * M_SHARD
            rows_ccw = q_ccw * M_SHARD + HALF

            rcv_cw = remote_copy(xg_ref.at[pl.ds(rows_cw, HALF), :],
                                 rows_cw, CW, h, right)
            rcv_ccw = remote_copy(xg_ref.at[pl.ds(rows_ccw, HALF), :],
                                  rows_ccw, CCW, h, left)
            rcv_cw.wait_recv()
            rcv_ccw.wait_recv()

            if h < N_DEV - 2:
                fcw = remote_copy(xg_ref.at[pl.ds(rows_cw, HALF), :],
                                  rows_cw, CW, h + 1, right)
                fccw = remote_copy(xg_ref.at[pl.ds(rows_ccw, HALF), :],
                                   rows_ccw, CCW, h + 1, left)
                fcw.start()
                fccw.start()
                sends += [fcw, fccw]

            chunk_gemm(xg_ref, rows_cw, rows_cw, HALF // BLK)
            chunk_gemm(xg_ref, rows_ccw, rows_ccw, HALF // BLK)

        for s in sends:
            s.wait_send()

    return pl.pallas_call(
        body,
        out_shape=jax.ShapeDtypeStruct((M_GLOBAL, N_SHARD), jnp.float32),
        in_specs=[
            pl.BlockSpec(memory_space=pl.MemorySpace.ANY),
            pl.BlockSpec(memory_space=pltpu.MemorySpace.VMEM),
        ],
        out_specs=pl.BlockSpec(memory_space=pl.MemorySpace.ANY),
        scratch_shapes=[
            pltpu.MemorySpace.HBM((M_GLOBAL, K), jnp.float32),
            pltpu.VMEM((BLK, K), jnp.float32),
            pltpu.VMEM((BLK, N_SHARD), jnp.float32),
            pltpu.SemaphoreType.DMA((2, N_DEV - 1)),
            pltpu.SemaphoreType.DMA((2, N_DEV - 1)),
            pltpu.SemaphoreType.DMA,
            pltpu.SemaphoreType.DMA,
        ],
        compiler_params=pltpu.CompilerParams(collective_id=0),
    )(x, w_mat)


# baseline (device time: 1189777 ns/iter reference)
import jax
import jax.numpy as jnp
from jax import lax
from jax.experimental import pallas as pl
from jax.experimental.pallas import tpu as pltpu

N_DEV = 4
M_GLOBAL = 8192
K = 8192
N_SHARD = 1024
M_SHARD = 2048
HALF = 1024
BLK = 512

CW, CCW = 0, 1


def kernel(x, w_mat):
    assert x.shape == (M_SHARD, K), x.shape
    assert w_mat.shape == (K, N_SHARD), w_mat.shape

    def body(x_ref, w_ref, out_ref, xg_ref, a_stage, o_stage,
             send_sems, recv_sems, cp_in_sem, cp_out_sem):
        p = lax.axis_index("i")
        left = lax.rem(p + N_DEV - 1, N_DEV)
        right = lax.rem(p + 1, N_DEV)

        barrier = pltpu.get_barrier_semaphore()
        for nbr in (left, right):
            pl.semaphore_signal(
                barrier, inc=1,
                device_id=(nbr,), device_id_type=pl.DeviceIdType.MESH,
            )
        pl.semaphore_wait(barrier, 2)

        def chunk_gemm(src_ref, src_base, out_base, nblk):
            def blk(b, carry):
                sr = src_base + b * BLK
                orow = out_base + b * BLK
                cp = pltpu.make_async_copy(
                    src_ref.at[pl.ds(sr, BLK), :], a_stage, cp_in_sem)
                cp.start()
                cp.wait()
                o_stage[...] = jnp.maximum(
                    jnp.dot(a_stage[...], w_ref[...],
                            preferred_element_type=jnp.float32),
                    0.0)
                cp2 = pltpu.make_async_copy(
                    o_stage, out_ref.at[pl.ds(orow, BLK), :], cp_out_sem)
                cp2.start()
                cp2.wait()
                return carry
            lax.fori_loop(0, nblk, blk, 0)

        def remote_copy(src_ref, rows, d, h, target):
            return pltpu.make_async_remote_copy(
                src_ref=src_ref,
                dst_ref=xg_ref.at[pl.ds(rows, HALF), :],
                send_sem=send_sems.at[d, h],
                recv_sem=recv_sems.at[d, h],
                device_id=(target,),
                device_id_type=pl.DeviceIdType.MESH,
            )

        sends = []

        cw0 = remote_copy(x_ref.at[pl.ds(0, HALF), :],
                          p * M_SHARD, CW, 0, right)
        ccw0 = remote_copy(x_ref.at[pl.ds(HALF, HALF), :],
                           p * M_SHARD + HALF, CCW, 0, left)
        cw0.start()
        ccw0.start()
        sends += [cw0, ccw0]

        chunk_gemm(x_ref, 0, p * M_SHARD, M_SHARD // BLK)

        for h in range(N_DEV - 1):
            q_cw = lax.rem(p + N_DEV - 1 - h, N_DEV)
            q_ccw = lax.rem(p + 1 + h, N_DEV)
            rows_cw = q_cw * M_SHARD
            rows_ccw = q_ccw * M_SHARD + HALF

            rcv_cw = remote_copy(xg_ref.at[pl.ds(rows_cw, HALF), :],
                                 rows_cw, CW, h, right)
            rcv_ccw = remote_copy(xg_ref.at[pl.ds(rows_ccw, HALF), :],
                                  rows_ccw, CCW, h, left)
            rcv_cw.wait_recv()
            rcv_ccw.wait_recv()

            if h < N_DEV - 2:
                fcw = remote_copy(xg_ref.at[pl.ds(rows_cw, HALF), :],
                                  rows_cw, CW, h + 1, right)
                fccw = remote_copy(xg_ref.at[pl.ds(rows_ccw, HALF), :],
                                   rows_ccw, CCW, h + 1, left)
                fcw.start()
                fccw.start()
                sends += [fcw, fccw]

            chunk_gemm(xg_ref, rows_cw, rows_cw, HALF // BLK)
            chunk_gemm(xg_ref, rows_ccw, rows_ccw, HALF // BLK)

        for s in sends:
            s.wait_send()

    out, _xg = pl.pallas_call(
        body,
        out_shape=[
            jax.ShapeDtypeStruct((M_GLOBAL, N_SHARD), jnp.float32),
            jax.ShapeDtypeStruct((M_GLOBAL, K), jnp.float32),
        ],
        in_specs=[
            pl.BlockSpec(memory_space=pl.MemorySpace.ANY),
            pl.BlockSpec(memory_space=pltpu.MemorySpace.VMEM),
        ],
        out_specs=[
            pl.BlockSpec(memory_space=pl.MemorySpace.ANY),
            pl.BlockSpec(memory_space=pl.MemorySpace.ANY),
        ],
        scratch_shapes=[
            pltpu.VMEM((BLK, K), jnp.float32),
            pltpu.VMEM((BLK, N_SHARD), jnp.float32),
            pltpu.SemaphoreType.DMA((2, N_DEV - 1)),
            pltpu.SemaphoreType.DMA((2, N_DEV - 1)),
            pltpu.SemaphoreType.DMA,
            pltpu.SemaphoreType.DMA,
        ],
        compiler_params=pltpu.CompilerParams(
            collective_id=0,
            vmem_limit_bytes=60 * 1024 * 1024,
        ),
    )(x, w_mat)
    return out
